# baseline (speedup 1.0000x reference)
import functools

import jax
import jax.numpy as jnp
from jax import lax
from jax.experimental import pallas as pl
from jax.experimental.pallas import tpu as pltpu
from jax.experimental.pallas import tpu_sc as plsc

_B, _S, _H, _V = 4, 2048, 1024, 250002
_TS = 256
_NT = _S // _TS
_VPAD = 253952
_NC, _NS = 2, 16
_WPS = 2 * _VPAD // _NS
_ZB = 8192
_RPT = (_B * _S) // (_NC * _NS * 128)


def _mv_body(mask_ref, hid_ref, wmv_ref, bmv_ref, wsp_ref, bsp_ref,
             wd_ref, bd_ref, mv_ref, tw_ref, dense_ref,
             pooled_scr, msum_scr):
    b = pl.program_id(0)
    st = pl.program_id(1)
    h = hid_ref[0]
    m = mask_ref[0, 0]

    mv = lax.dot_general(h, wmv_ref[...], (((1,), (1,)), ((), ())),
                         preferred_element_type=jnp.float32)
    mv = (mv + bmv_ref[...]) * m
    n = jnp.sqrt(jnp.sum(mv * mv, axis=-1, keepdims=True))
    mv_ref[0] = mv / jnp.maximum(n, 1e-12)

    tw = jnp.sum(h * wsp_ref[...], axis=-1, keepdims=True) + bsp_ref[...]
    tw = jnp.maximum(tw * m, 0.0)
    tw_ref[0, 0] = jnp.where(m > 0, tw, 0.0)

    part = jnp.sum(h * m, axis=0, keepdims=True)
    ms = jnp.full((1, 128), jnp.sum(m), jnp.float32)

    @pl.when(st == 0)
    def _():
        pooled_scr[pl.ds(b, 1), :] = part
        msum_scr[pl.ds(b, 1), :] = ms

    @pl.when(st != 0)
    def _():
        pooled_scr[pl.ds(b, 1), :] += part
        msum_scr[pl.ds(b, 1), :] += ms

    @pl.when(jnp.logical_and(b == _B - 1, st == _NT - 1))
    def _():
        p = pooled_scr[...]
        msv = msum_scr[...][:, 0:1]
        d = lax.dot_general(p / msv, wd_ref[...], (((1,), (1,)), ((), ())),
                            preferred_element_type=jnp.float32)
        d = jnp.tanh(d + bd_ref[...])
        dn = jnp.sqrt(jnp.sum(d * d, axis=-1, keepdims=True))
        dense_ref[...] = d / jnp.maximum(dn, 1e-12)


def _scatter_body(ids_hbm, w_hbm, out_hbm, idx_v, w_v, zbuf, acc):
    c = lax.axis_index("c")
    s = lax.axis_index("s")

    def zbuf_body(i, carry):
        zbuf[pl.ds(i * 16, 16)] = jnp.zeros((16,), jnp.float32)
        return carry

    lax.fori_loop(0, _ZB // 16, zbuf_body, 0, unroll=8)

    base_row = (c * _NS + s) * _RPT
    zbase = s * _WPS
    pltpu.sync_copy(ids_hbm.at[pl.ds(base_row, _RPT)], idx_v)
    pltpu.sync_copy(w_hbm.at[pl.ds(base_row, _RPT)], w_v)
    off_w = 0
    while off_w < _WPS:
        n = min(_ZB, _WPS - off_w)
        pltpu.sync_copy(zbuf.at[pl.ds(0, n)], acc.at[pl.ds(zbase + off_w, n)])
        off_w += n

    off = (s // (_NS // 2)) * _VPAD
    for j in range(_RPT):
        for i in range(128 // 16):
            idx_v[j, pl.ds(i * 16, 16)] = idx_v[j, pl.ds(i * 16, 16)] + off
    plsc.subcore_barrier()
    for j in range(_RPT):
        pltpu.sync_copy(w_v.at[j], acc.at[idx_v.at[j]], add=True)

    plsc.subcore_barrier()
    pltpu.sync_copy(acc.at[pl.ds(s * _WPS, _WPS)],
                    out_hbm.at[pl.ds(c * 2 * _VPAD + s * _WPS, _WPS)])


@functools.cache
def _scatter_kernel():
    return pl.kernel(
        _scatter_body,
        out_type=jax.ShapeDtypeStruct((_B * _VPAD,), jnp.float32),
        mesh=plsc.VectorSubcoreMesh(core_axis_name="c", subcore_axis_name="s",
                                    num_cores=_NC, num_subcores=_NS),
        scratch_types=[
            pltpu.VMEM((_RPT, 128), jnp.int32),
            pltpu.VMEM((_RPT, 128), jnp.float32),
            pltpu.VMEM((_ZB,), jnp.float32),
            pltpu.VMEM_SHARED((2 * _VPAD,), jnp.float32),
        ],
    )


def _tc_heads(hidden_states, mask4, W_mv, b_mv, W_sparse, b_sparse,
              W_dense, b_dense):
    f32 = jnp.float32
    return pl.pallas_call(
        _mv_body,
        grid=(_B, _NT),
        in_specs=[
            pl.BlockSpec((1, 1, _TS, 1), lambda b, st: (b, st, 0, 0)),
            pl.BlockSpec((1, _TS, _H), lambda b, st: (b, st, 0)),
            pl.BlockSpec((_H, _H), lambda b, st: (0, 0)),
            pl.BlockSpec((1, _H), lambda b, st: (0, 0)),
            pl.BlockSpec((1, _H), lambda b, st: (0, 0)),
            pl.BlockSpec((1, 1), lambda b, st: (0, 0)),
            pl.BlockSpec((_H, _H), lambda b, st: (0, 0)),
            pl.BlockSpec((1, _H), lambda b, st: (0, 0)),
        ],
        out_specs=[
            pl.BlockSpec((1, _TS, _H), lambda b, st: (b, st, 0)),
            pl.BlockSpec((1, 1, _TS, 1), lambda b, st: (b, st, 0, 0)),
            pl.BlockSpec((_B, _H), lambda b, st: (0, 0)),
        ],
        out_shape=[
            jax.ShapeDtypeStruct((_B, _S, _H), f32),
            jax.ShapeDtypeStruct((_B, _NT, _TS, 1), f32),
            jax.ShapeDtypeStruct((_B, _H), f32),
        ],
        scratch_shapes=[
            pltpu.VMEM((_B, _H), f32),
            pltpu.VMEM((_B, 128), f32),
        ],
    )(mask4, hidden_states, W_mv, b_mv, W_sparse, b_sparse, W_dense, b_dense)


def kernel(hidden_states, input_ids, attention_mask, W_dense, b_dense,
           W_sparse, b_sparse, W_mv, b_mv):
    f32 = jnp.float32
    hs = hidden_states.astype(f32)
    mask = attention_mask.astype(f32)
    mask4 = mask.reshape(_B, _NT, _TS, 1)

    mv, tw, dense = _tc_heads(
        hs, mask4, W_mv.astype(f32), b_mv.astype(f32).reshape(1, _H),
        W_sparse.astype(f32), b_sparse.astype(f32).reshape(1, 1),
        W_dense.astype(f32), b_dense.astype(f32).reshape(1, _H))

    ids2d = input_ids.astype(jnp.int32).reshape(-1, 128)
    w2d = tw.reshape(-1, 128)
    sp_flat = _scatter_kernel()(ids2d, w2d)
    sparse = sp_flat.reshape(_B, _VPAD)[:, :_V]

    return dense, sparse, mv


# speedup vs baseline: 1.1882x; 1.1882x over previous
import functools

import jax
import jax.numpy as jnp
from jax import lax
from jax.experimental import pallas as pl
from jax.experimental.pallas import tpu as pltpu
from jax.experimental.pallas import tpu_sc as plsc

_B, _S, _H, _V = 4, 2048, 1024, 250002
_TS = 1024
_NT = _S // _TS
_VPAD = 253952
_NC, _NS = 2, 16
_WPS = 2 * _VPAD // _NS
_ZB = 8192
_RPT = (_B * _S) // (_NC * _NS * 128)


def _mv_body(mask_ref, hid_ref, wmv_ref, bmv_ref, wsp_ref, bsp_ref,
             wd_ref, bd_ref, mv_ref, tw_ref, dense_ref,
             pooled_scr, msum_scr):
    b = pl.program_id(0)
    st = pl.program_id(1)
    h = hid_ref[0]
    m = mask_ref[0, 0]

    mv = lax.dot_general(h, wmv_ref[...], (((1,), (1,)), ((), ())),
                         preferred_element_type=jnp.float32)
    mv = (mv + bmv_ref[...]) * m
    n = jnp.sqrt(jnp.sum(mv * mv, axis=-1, keepdims=True))
    mv_ref[0] = mv / jnp.maximum(n, 1e-12)

    tw = jnp.sum(h * wsp_ref[...], axis=-1, keepdims=True) + bsp_ref[...]
    tw = jnp.maximum(tw * m, 0.0)
    tw_ref[0, 0] = jnp.where(m > 0, tw, 0.0)

    part = jnp.sum(h * m, axis=0, keepdims=True)
    ms = jnp.full((1, 128), jnp.sum(m), jnp.float32)

    @pl.when(st == 0)
    def _():
        pooled_scr[pl.ds(b, 1), :] = part
        msum_scr[pl.ds(b, 1), :] = ms

    @pl.when(st != 0)
    def _():
        pooled_scr[pl.ds(b, 1), :] += part
        msum_scr[pl.ds(b, 1), :] += ms

    @pl.when(jnp.logical_and(b == _B - 1, st == _NT - 1))
    def _():
        p = pooled_scr[...]
        msv = msum_scr[...][:, 0:1]
        d = lax.dot_general(p / msv, wd_ref[...], (((1,), (1,)), ((), ())),
                            preferred_element_type=jnp.float32)
        d = jnp.tanh(d + bd_ref[...])
        dn = jnp.sqrt(jnp.sum(d * d, axis=-1, keepdims=True))
        dense_ref[...] = d / jnp.maximum(dn, 1e-12)


def _scatter_body(ids_hbm, w_hbm, out_hbm, idx_v, w_v, zbuf, acc):
    c = lax.axis_index("c")
    s = lax.axis_index("s")

    def zbuf_body(i, carry):
        zbuf[pl.ds(i * 16, 16)] = jnp.zeros((16,), jnp.float32)
        return carry

    lax.fori_loop(0, _ZB // 16, zbuf_body, 0, unroll=8)

    base_row = (c * _NS + s) * _RPT
    zbase = s * _WPS
    pltpu.sync_copy(ids_hbm.at[pl.ds(base_row, _RPT)], idx_v)
    pltpu.sync_copy(w_hbm.at[pl.ds(base_row, _RPT)], w_v)
    off_w = 0
    while off_w < _WPS:
        n = min(_ZB, _WPS - off_w)
        pltpu.sync_copy(zbuf.at[pl.ds(0, n)], acc.at[pl.ds(zbase + off_w, n)])
        off_w += n

    off = (s // (_NS // 2)) * _VPAD
    for j in range(_RPT):
        for i in range(128 // 16):
            idx_v[j, pl.ds(i * 16, 16)] = idx_v[j, pl.ds(i * 16, 16)] + off
    plsc.subcore_barrier()
    for j in range(_RPT):
        pltpu.sync_copy(w_v.at[j], acc.at[idx_v.at[j]], add=True)

    plsc.subcore_barrier()
    pltpu.sync_copy(acc.at[pl.ds(s * _WPS, _WPS)],
                    out_hbm.at[pl.ds(c * 2 * _VPAD + s * _WPS, _WPS)])


@functools.cache
def _scatter_kernel():
    return pl.kernel(
        _scatter_body,
        out_type=jax.ShapeDtypeStruct((_B * _VPAD,), jnp.float32),
        mesh=plsc.VectorSubcoreMesh(core_axis_name="c", subcore_axis_name="s",
                                    num_cores=_NC, num_subcores=_NS),
        scratch_types=[
            pltpu.VMEM((_RPT, 128), jnp.int32),
            pltpu.VMEM((_RPT, 128), jnp.float32),
            pltpu.VMEM((_ZB,), jnp.float32),
            pltpu.VMEM_SHARED((2 * _VPAD,), jnp.float32),
        ],
    )


def _tc_heads(hidden_states, mask4, W_mv, b_mv, W_sparse, b_sparse,
              W_dense, b_dense):
    f32 = jnp.float32
    return pl.pallas_call(
        _mv_body,
        grid=(_B, _NT),
        in_specs=[
            pl.BlockSpec((1, 1, _TS, 1), lambda b, st: (b, st, 0, 0)),
            pl.BlockSpec((1, _TS, _H), lambda b, st: (b, st, 0)),
            pl.BlockSpec((_H, _H), lambda b, st: (0, 0)),
            pl.BlockSpec((1, _H), lambda b, st: (0, 0)),
            pl.BlockSpec((1, _H), lambda b, st: (0, 0)),
            pl.BlockSpec((1, 1), lambda b, st: (0, 0)),
            pl.BlockSpec((_H, _H), lambda b, st: (0, 0)),
            pl.BlockSpec((1, _H), lambda b, st: (0, 0)),
        ],
        out_specs=[
            pl.BlockSpec((1, _TS, _H), lambda b, st: (b, st, 0)),
            pl.BlockSpec((1, 1, _TS, 1), lambda b, st: (b, st, 0, 0)),
            pl.BlockSpec((_B, _H), lambda b, st: (0, 0)),
        ],
        out_shape=[
            jax.ShapeDtypeStruct((_B, _S, _H), f32),
            jax.ShapeDtypeStruct((_B, _NT, _TS, 1), f32),
            jax.ShapeDtypeStruct((_B, _H), f32),
        ],
        scratch_shapes=[
            pltpu.VMEM((_B, _H), f32),
            pltpu.VMEM((_B, 128), f32),
        ],
    )(mask4, hidden_states, W_mv, b_mv, W_sparse, b_sparse, W_dense, b_dense)


def kernel(hidden_states, input_ids, attention_mask, W_dense, b_dense,
           W_sparse, b_sparse, W_mv, b_mv):
    f32 = jnp.float32
    hs = hidden_states.astype(f32)
    mask = attention_mask.astype(f32)
    mask4 = mask.reshape(_B, _NT, _TS, 1)

    mv, tw, dense = _tc_heads(
        hs, mask4, W_mv.astype(f32), b_mv.astype(f32).reshape(1, _H),
        W_sparse.astype(f32), b_sparse.astype(f32).reshape(1, 1),
        W_dense.astype(f32), b_dense.astype(f32).reshape(1, _H))

    ids2d = input_ids.astype(jnp.int32).reshape(-1, 128)
    w2d = tw.reshape(-1, 128)
    sp_flat = _scatter_kernel()(ids2d, w2d)
    sparse = sp_flat.reshape(_B, _VPAD)[:, :_V]

    return dense, sparse, mv


# speedup vs baseline: 1.2208x; 1.0274x over previous
import functools

import jax
import jax.numpy as jnp
from jax import lax
from jax.experimental import pallas as pl
from jax.experimental.pallas import tpu as pltpu
from jax.experimental.pallas import tpu_sc as plsc

_B, _S, _H, _V = 4, 2048, 1024, 250002
_TS = 2048
_NT = _S // _TS
_VPAD = 253952
_NC, _NS = 2, 16
_WPS = 2 * _VPAD // _NS
_ZB = 8192
_RPT = (_B * _S) // (_NC * _NS * 128)


def _mv_body(mask_ref, hid_ref, wmv_ref, bmv_ref, wsp_ref, bsp_ref,
             wd_ref, bd_ref, mv_ref, tw_ref, dense_ref,
             pooled_scr, msum_scr):
    b = pl.program_id(0)
    st = pl.program_id(1)
    h = hid_ref[0]
    m = mask_ref[0, 0]

    mv = lax.dot_general(h, wmv_ref[...], (((1,), (1,)), ((), ())),
                         preferred_element_type=jnp.float32)
    mv = (mv + bmv_ref[...]) * m
    n = jnp.sqrt(jnp.sum(mv * mv, axis=-1, keepdims=True))
    mv_ref[0] = mv / jnp.maximum(n, 1e-12)

    tw = jnp.sum(h * wsp_ref[...], axis=-1, keepdims=True) + bsp_ref[...]
    tw = jnp.maximum(tw * m, 0.0)
    tw_ref[0, 0] = jnp.where(m > 0, tw, 0.0)

    part = jnp.sum(h * m, axis=0, keepdims=True)
    ms = jnp.full((1, 128), jnp.sum(m), jnp.float32)

    @pl.when(st == 0)
    def _():
        pooled_scr[pl.ds(b, 1), :] = part
        msum_scr[pl.ds(b, 1), :] = ms

    @pl.when(st != 0)
    def _():
        pooled_scr[pl.ds(b, 1), :] += part
        msum_scr[pl.ds(b, 1), :] += ms

    @pl.when(jnp.logical_and(b == _B - 1, st == _NT - 1))
    def _():
        p = pooled_scr[...]
        msv = msum_scr[...][:, 0:1]
        d = lax.dot_general(p / msv, wd_ref[...], (((1,), (1,)), ((), ())),
                            preferred_element_type=jnp.float32)
        d = jnp.tanh(d + bd_ref[...])
        dn = jnp.sqrt(jnp.sum(d * d, axis=-1, keepdims=True))
        dense_ref[...] = d / jnp.maximum(dn, 1e-12)


def _scatter_body(ids_hbm, w_hbm, out_hbm, idx_v, w_v, zbuf, acc):
    c = lax.axis_index("c")
    s = lax.axis_index("s")

    def zbuf_body(i, carry):
        zbuf[pl.ds(i * 16, 16)] = jnp.zeros((16,), jnp.float32)
        return carry

    lax.fori_loop(0, _ZB // 16, zbuf_body, 0, unroll=8)

    base_row = (c * _NS + s) * _RPT
    zbase = s * _WPS
    pltpu.sync_copy(ids_hbm.at[pl.ds(base_row, _RPT)], idx_v)
    pltpu.sync_copy(w_hbm.at[pl.ds(base_row, _RPT)], w_v)
    off_w = 0
    while off_w < _WPS:
        n = min(_ZB, _WPS - off_w)
        pltpu.sync_copy(zbuf.at[pl.ds(0, n)], acc.at[pl.ds(zbase + off_w, n)])
        off_w += n

    off = (s // (_NS // 2)) * _VPAD
    for j in range(_RPT):
        for i in range(128 // 16):
            idx_v[j, pl.ds(i * 16, 16)] = idx_v[j, pl.ds(i * 16, 16)] + off
    plsc.subcore_barrier()
    for j in range(_RPT):
        pltpu.sync_copy(w_v.at[j], acc.at[idx_v.at[j]], add=True)

    plsc.subcore_barrier()
    pltpu.sync_copy(acc.at[pl.ds(s * _WPS, _WPS)],
                    out_hbm.at[pl.ds(c * 2 * _VPAD + s * _WPS, _WPS)])


@functools.cache
def _scatter_kernel():
    return pl.kernel(
        _scatter_body,
        out_type=jax.ShapeDtypeStruct((_B * _VPAD,), jnp.float32),
        mesh=plsc.VectorSubcoreMesh(core_axis_name="c", subcore_axis_name="s",
                                    num_cores=_NC, num_subcores=_NS),
        scratch_types=[
            pltpu.VMEM((_RPT, 128), jnp.int32),
            pltpu.VMEM((_RPT, 128), jnp.float32),
            pltpu.VMEM((_ZB,), jnp.float32),
            pltpu.VMEM_SHARED((2 * _VPAD,), jnp.float32),
        ],
    )


def _tc_heads(hidden_states, mask4, W_mv, b_mv, W_sparse, b_sparse,
              W_dense, b_dense):
    f32 = jnp.float32
    return pl.pallas_call(
        _mv_body,
        grid=(_B, _NT),
        in_specs=[
            pl.BlockSpec((1, 1, _TS, 1), lambda b, st: (b, st, 0, 0)),
            pl.BlockSpec((1, _TS, _H), lambda b, st: (b, st, 0)),
            pl.BlockSpec((_H, _H), lambda b, st: (0, 0)),
            pl.BlockSpec((1, _H), lambda b, st: (0, 0)),
            pl.BlockSpec((1, _H), lambda b, st: (0, 0)),
            pl.BlockSpec((1, 1), lambda b, st: (0, 0)),
            pl.BlockSpec((_H, _H), lambda b, st: (0, 0)),
            pl.BlockSpec((1, _H), lambda b, st: (0, 0)),
        ],
        out_specs=[
            pl.BlockSpec((1, _TS, _H), lambda b, st: (b, st, 0)),
            pl.BlockSpec((1, 1, _TS, 1), lambda b, st: (b, st, 0, 0)),
            pl.BlockSpec((_B, _H), lambda b, st: (0, 0)),
        ],
        out_shape=[
            jax.ShapeDtypeStruct((_B, _S, _H), f32),
            jax.ShapeDtypeStruct((_B, _NT, _TS, 1), f32),
            jax.ShapeDtypeStruct((_B, _H), f32),
        ],
        scratch_shapes=[
            pltpu.VMEM((_B, _H), f32),
            pltpu.VMEM((_B, 128), f32),
        ],
    )(mask4, hidden_states, W_mv, b_mv, W_sparse, b_sparse, W_dense, b_dense)


def kernel(hidden_states, input_ids, attention_mask, W_dense, b_dense,
           W_sparse, b_sparse, W_mv, b_mv):
    f32 = jnp.float32
    hs = hidden_states.astype(f32)
    mask = attention_mask.astype(f32)
    mask4 = mask.reshape(_B, _NT, _TS, 1)

    mv, tw, dense = _tc_heads(
        hs, mask4, W_mv.astype(f32), b_mv.astype(f32).reshape(1, _H),
        W_sparse.astype(f32), b_sparse.astype(f32).reshape(1, 1),
        W_dense.astype(f32), b_dense.astype(f32).reshape(1, _H))

    ids2d = input_ids.astype(jnp.int32).reshape(-1, 128)
    w2d = tw.reshape(-1, 128)
    sp_flat = _scatter_kernel()(ids2d, w2d)
    sparse = sp_flat.reshape(_B, _VPAD)[:, :_V]

    return dense, sparse, mv
